# baseline (speedup 1.0000x reference)
import jax
import jax.numpy as jnp
from jax import lax
from jax.experimental import pallas as pl
from jax.experimental.pallas import tpu as pltpu

N = 32768
D = 128
K = 512
BN = 4096
NBLK = N // (2 * BN)


def _half(a, c, cm, idx_ref, lo):
    u = lax.dot_general(c, a, (((1,), (1,)), ((), ())),
                        preferred_element_type=jnp.float32) + cm
    m = jnp.max(u, axis=0, keepdims=True)
    row = lax.broadcasted_iota(jnp.int32, (K, BN), 0).astype(jnp.float32)
    idx = jnp.min(jnp.where(u >= m, row, float(K)), axis=0)
    idx_ref[pl.ds(lo, BN)] = idx.astype(jnp.int32)
    return jnp.sum(a * a) - 2.0 * jnp.sum(m)


def _body(a0_ref, a1_ref, c_ref, idx_ref, loss_ref, cm_ref, acc_ref):
    @pl.when(pl.program_id(0) == 0)
    def _prep():
        c0 = c_ref[...]
        cm_ref[...] = -0.5 * jnp.sum(c0 * c0, axis=1, keepdims=True)
        acc_ref[0] = 0.0

    c = c_ref[...]
    cm = cm_ref[...]
    p0 = _half(a0_ref[...], c, cm, idx_ref, 0)
    p1 = _half(a1_ref[...], c, cm, idx_ref, BN)
    acc_ref[0] += p0 + p1

    @pl.when(pl.program_id(0) == NBLK - 1)
    def _fin():
        loss_ref[0] = acc_ref[0] * (1.0 / (N * D))


@jax.jit
def _run(embeddings, centroids):
    idx, loss = pl.pallas_call(
        _body,
        grid=(NBLK,),
        in_specs=[
            pl.BlockSpec((BN, D), lambda i: (2 * i, 0)),
            pl.BlockSpec((BN, D), lambda i: (2 * i + 1, 0)),
            pl.BlockSpec((K, D), lambda i: (0, 0)),
        ],
        out_specs=[
            pl.BlockSpec((2 * BN,), lambda i: (i,)),
            pl.BlockSpec(memory_space=pltpu.SMEM),
        ],
        out_shape=[
            jax.ShapeDtypeStruct((N,), jnp.int32),
            jax.ShapeDtypeStruct((1,), jnp.float32),
        ],
        scratch_shapes=[
            pltpu.VMEM((K, 1), jnp.float32),
            pltpu.SMEM((1,), jnp.float32),
        ],
    )(embeddings, embeddings, centroids)
    return idx, loss


def kernel(embeddings, centroids):
    idx, loss = _run(embeddings, centroids)
    return (loss.reshape(()), idx)
